# baseline (speedup 1.0000x reference)
import functools

import jax
import jax.numpy as jnp
from jax import lax
from jax.experimental import pallas as pl
from jax.experimental.pallas import tpu as pltpu
from jax.experimental.pallas import tpu_sc as plsc

B, M, L, W, H = 4096, 200, 24, 11, 11
CNN_CH, HID = 128, 512
LP = 128
C_IN = L * LP
NPOS = 9
C_MID = CNN_CH * NPOS
BB = 256

NC, NS, NW = 2, 16, 32
SPW = B // NW
G = 8
NG = SPW // G


def _sc_scatter(obs_t):
    mesh = plsc.VectorSubcoreMesh(core_axis_name="c", subcore_axis_name="s")

    @functools.partial(
        pl.kernel,
        out_type=jax.ShapeDtypeStruct((B, C_IN), jnp.float32),
        mesh=mesh,
        scratch_types=[
            pltpu.VMEM((3, M, SPW), jnp.int32),
            pltpu.VMEM((2, G, C_IN), jnp.float32),
            pltpu.SemaphoreType.DMA,
            pltpu.SemaphoreType.DMA,
        ],
        compiler_params=pltpu.CompilerParams(needs_layout_passes=False),
    )
    def sck(obs_hbm, box_hbm, obs_v, box_v, osem0, osem1):
        wid = lax.axis_index("s") * NC + lax.axis_index("c")
        base = wid * SPW
        lane = lax.iota(jnp.int32, 16)
        pat_s = lane & 7
        pat_t = lane >> 3
        fvec = lane * 0
        zv = jnp.zeros((16,), jnp.float32)

        pltpu.sync_copy(obs_hbm.at[:, :, pl.ds(base, SPW)], obs_v)

        def group(g, p, osem, not_first):
            @pl.when(not_first)
            def _():
                pltpu.make_async_copy(
                    box_v.at[p], box_hbm.at[pl.ds(base, G)], osem).wait()

            def zero(i, carry):
                for r in range(G):
                    box_v[p, r, pl.ds(i * 16, 16)] = zv
                return carry

            lax.fori_loop(0, C_IN // 16, zero, 0, unroll=2)
            s0 = g * G

            def token(t, carry):
                tv = pat_t + 2 * t
                sv = pat_s + s0
                cb = plsc.load_gather(obs_v, [fvec, tv, sv])
                at = plsc.load_gather(obs_v, [fvec + 1, tv, sv])
                vl = plsc.load_gather(obs_v, [fvec + 2, tv, sv])
                cb = jnp.where(cb == 255, 0, cb)
                at = jnp.where(at == 255, 0, at)
                vl = jnp.where(vl == 255, 0, vl)
                x = (cb >> 4) & 15
                y = cb & 15
                valid = (x < W) & (y < H) & (at < L)
                c = jnp.where(valid, at * LP + x * W + y, 0)
                plsc.store_scatter(box_v.at[p], [pat_s, c],
                                   vl.astype(jnp.float32), mask=valid)
                return carry

            lax.fori_loop(0, M // 2, token, 0, unroll=2)
            pltpu.make_async_copy(
                box_v.at[p], box_hbm.at[pl.ds(base + g * G, G)], osem).start()

        def pairs(u, carry):
            group(2 * u, 0, osem0, u > 0)
            group(2 * u + 1, 1, osem1, u > 0)
            return carry

        lax.fori_loop(0, NG // 2, pairs, 0, unroll=False)
        pltpu.make_async_copy(
            box_v.at[0], box_hbm.at[pl.ds(base, G)], osem0).wait()
        pltpu.make_async_copy(
            box_v.at[1], box_hbm.at[pl.ds(base, G)], osem1).wait()

    return sck(obs_t)


N_COLS = 10 * CNN_CH


def _build_wbig(conv1_w, max_vec):
    inv = 1.0 / (max_vec.reshape(L) + 1e-8)
    wt = conv1_w.transpose(1, 2, 3, 0)
    canvas = jnp.zeros((L, W, H, 10, CNN_CH), jnp.float32)
    for i in range(3):
        for j in range(3):
            canvas = canvas.at[:, 3 * i:3 * i + 5, 3 * j:3 * j + 5,
                               3 * i + j, :].set(wt)
    cen = jnp.zeros((L, W, H, CNN_CH), jnp.float32)
    cen = cen.at[jnp.arange(L), 5, 5, jnp.arange(L)].set(
        jnp.ones((L,), jnp.float32))
    canvas = canvas.at[:, :, :, NPOS, :].set(cen)
    wb = canvas.reshape(L, W * H, N_COLS)
    wb = jnp.pad(wb, ((0, 0), (0, LP - W * H), (0, 0)))
    wb = wb.reshape(C_IN, N_COLS)
    invr = jnp.repeat(inv, LP)
    return (wb * invr[:, None]).astype(jnp.bfloat16)


def _dense_body(box_ref, wbig_ref, w2_ref, fc_ref, se_ref,
                b1_ref, b2_ref, bfc_ref, bse_ref,
                a0s_ref, a0c_ref, a1s_ref, a1c_ref, vs_ref, vc_ref,
                a0b_ref, a1b_ref, vb_ref,
                o0_ref, o1_ref, ov_ref):
    f32 = jnp.float32
    box = box_ref[...].astype(jnp.bfloat16)
    acc = jnp.dot(box, wbig_ref[...], preferred_element_type=f32)
    conv1 = jnp.maximum(acc[:, :C_MID] + b1_ref[...], 0.0)
    center = acc[:, C_MID:C_MID + L]
    selff = jnp.maximum(
        jnp.dot(center, se_ref[...], preferred_element_type=f32)
        + bse_ref[...], 0.0)
    h2 = jnp.maximum(
        jnp.dot(conv1, w2_ref[...], preferred_element_type=f32)
        + b2_ref[...], 0.0)
    cnn = jnp.maximum(
        jnp.dot(h2, fc_ref[...], preferred_element_type=f32)
        + bfc_ref[...], 0.0)
    o0_ref[...] = (jnp.dot(selff, a0s_ref[...], preferred_element_type=f32)
                   + jnp.dot(cnn, a0c_ref[...], preferred_element_type=f32)
                   + a0b_ref[...])
    o1_ref[...] = (jnp.dot(selff, a1s_ref[...], preferred_element_type=f32)
                   + jnp.dot(cnn, a1c_ref[...], preferred_element_type=f32)
                   + a1b_ref[...])
    ov_ref[...] = (jnp.dot(selff, vs_ref[...], preferred_element_type=f32)
                   + jnp.dot(cnn, vc_ref[...], preferred_element_type=f32)
                   + vb_ref[...])


def _dense_stage(box, wbig, w2r, fc_t, se_t, b1r, conv2_b, fc_b, se_b,
                 a0s, a0c, a1s, a1c, vs, vc, a0_b, a1_b, v_b, interpret=False):
    nb = B // BB
    full = lambda shp: pl.BlockSpec(shp, lambda i: (0,) * len(shp))
    grid_spec = pl.GridSpec(
        grid=(nb,),
        in_specs=[
            pl.BlockSpec((BB, C_IN), lambda i: (i, 0)),
            full((C_IN, N_COLS)), full((C_MID, CNN_CH)),
            full((CNN_CH, HID // 2)), full((L, HID // 2)),
            full((1, C_MID)), full((1, CNN_CH)), full((1, HID // 2)),
            full((1, HID // 2)),
            full((HID // 2, 128)), full((HID // 2, 128)),
            full((HID // 2, 256)), full((HID // 2, 256)),
            full((HID // 2, 128)), full((HID // 2, 128)),
            full((1, 128)), full((1, 256)), full((1, 128)),
        ],
        out_specs=[
            pl.BlockSpec((BB, 128), lambda i: (i, 0)),
            pl.BlockSpec((BB, 256), lambda i: (i, 0)),
            pl.BlockSpec((BB, 128), lambda i: (i, 0)),
        ],
    )
    o0, o1, ov = pl.pallas_call(
        _dense_body,
        grid_spec=grid_spec,
        out_shape=[
            jax.ShapeDtypeStruct((B, 128), jnp.float32),
            jax.ShapeDtypeStruct((B, 256), jnp.float32),
            jax.ShapeDtypeStruct((B, 128), jnp.float32),
        ],
        compiler_params=pltpu.CompilerParams(
            dimension_semantics=("parallel",)),
        interpret=interpret,
    )(box, wbig, w2r, fc_t, se_t, b1r, conv2_b, fc_b, se_b,
      a0s, a0c, a1s, a1c, vs, vc, a0_b, a1_b, v_b)
    return o0[:, :9], o1, ov[:, :1]


def kernel(observations, conv1_w, conv1_b, conv2_w, conv2_b, fc_w, fc_b,
           se_w, se_b, a0_w, a0_b, a1_w, a1_b, v_w, v_b, max_vec):
    obs_t = jnp.transpose(observations, (2, 1, 0))
    box = _sc_scatter(obs_t)

    wbig = _build_wbig(conv1_w, max_vec)
    w2r = conv2_w.transpose(2, 3, 1, 0).reshape(C_MID, CNN_CH)
    fc_t = fc_w.T
    se_t = se_w.T
    b1r = jnp.tile(conv1_b, NPOS).reshape(1, C_MID)
    pad_h = lambda w, n: jnp.pad(w.T, ((0, 0), (0, n - w.shape[0])))
    a0s, a0c = pad_h(a0_w[:, :256], 128), pad_h(a0_w[:, 256:], 128)
    a1s, a1c = a1_w[:, :256].T, a1_w[:, 256:].T
    vs, vc = pad_h(v_w[:, :256], 128), pad_h(v_w[:, 256:], 128)
    a0b = jnp.pad(a0_b, (0, 128 - 9)).reshape(1, 128)
    a1b = a1_b.reshape(1, 256)
    vb = jnp.pad(v_b, (0, 127)).reshape(1, 128)

    return _dense_stage(box, wbig, w2r, fc_t, se_t, b1r,
                        conv2_b.reshape(1, CNN_CH), fc_b.reshape(1, HID // 2),
                        se_b.reshape(1, HID // 2),
                        a0s, a0c, a1s, a1c, vs, vc, a0b, a1b, vb)


# speedup vs baseline: 1.4453x; 1.4453x over previous
import functools

import jax
import jax.numpy as jnp
from jax import lax
from jax.experimental import pallas as pl
from jax.experimental.pallas import tpu as pltpu
from jax.experimental.pallas import tpu_sc as plsc

B, M, L, W, H = 4096, 200, 24, 11, 11
CNN_CH, HID = 128, 512
LP = 128
C_IN = L * LP
NPOS = 9
C_MID = CNN_CH * NPOS
BB = 256
N_COLS = 10 * CNN_CH

NC, NS, NW = 2, 16, 32
SPW = B // NW
G = 8
NG = SPW // G


def _sc_scatter(obs_t):
    mesh = plsc.VectorSubcoreMesh(core_axis_name="c", subcore_axis_name="s")

    @functools.partial(
        pl.kernel,
        out_type=jax.ShapeDtypeStruct((B, C_IN), jnp.float32),
        mesh=mesh,
        scratch_types=[
            pltpu.VMEM((3, M, SPW), jnp.int32),
            pltpu.VMEM((2, G, C_IN), jnp.float32),
            pltpu.SemaphoreType.DMA,
            pltpu.SemaphoreType.DMA,
        ],
        compiler_params=pltpu.CompilerParams(needs_layout_passes=False),
    )
    def sck(obs_hbm, box_hbm, obs_v, box_v, osem0, osem1):
        wid = lax.axis_index("s") * NC + lax.axis_index("c")
        base = wid * SPW
        lane = lax.iota(jnp.int32, 16)
        pat_s = lane & 7
        pat_t = lane >> 3
        fvec = lane * 0
        zv = jnp.zeros((16,), jnp.float32)

        pltpu.sync_copy(obs_hbm.at[:, :, pl.ds(base, SPW)], obs_v)

        def group(g, p, osem, not_first):
            @pl.when(not_first)
            def _():
                pltpu.make_async_copy(
                    box_v.at[p], box_hbm.at[pl.ds(base, G)], osem).wait()

            def zero(i, carry):
                for r in range(G):
                    box_v[p, r, pl.ds(i * 16, 16)] = zv
                return carry

            lax.fori_loop(0, C_IN // 16, zero, 0, unroll=2)
            s0 = g * G

            def token(t, carry):
                tv = pat_t + 2 * t
                sv = pat_s + s0
                cb = plsc.load_gather(obs_v, [fvec, tv, sv])
                at = plsc.load_gather(obs_v, [fvec + 1, tv, sv])
                vl = plsc.load_gather(obs_v, [fvec + 2, tv, sv])
                cb = jnp.where(cb == 255, 0, cb)
                at = jnp.where(at == 255, 0, at)
                vl = jnp.where(vl == 255, 0, vl)
                x = (cb >> 4) & 15
                y = cb & 15
                valid = (x < W) & (y < H) & (at < L)
                c = jnp.where(valid, at * LP + x * W + y, 0)
                plsc.store_scatter(box_v.at[p], [pat_s, c],
                                   vl.astype(jnp.float32), mask=valid)
                return carry

            lax.fori_loop(0, M // 2, token, 0, unroll=2)
            pltpu.make_async_copy(
                box_v.at[p], box_hbm.at[pl.ds(base + g * G, G)], osem).start()

        def pairs(u, carry):
            group(2 * u, 0, osem0, u > 0)
            group(2 * u + 1, 1, osem1, u > 0)
            return carry

        lax.fori_loop(0, NG // 2, pairs, 0, unroll=False)
        pltpu.make_async_copy(
            box_v.at[0], box_hbm.at[pl.ds(base, G)], osem0).wait()
        pltpu.make_async_copy(
            box_v.at[1], box_hbm.at[pl.ds(base, G)], osem1).wait()

    return sck(obs_t)


def _build_wbig(conv1_w, max_vec):
    inv = 1.0 / (max_vec.reshape(L) + 1e-8)
    wt = conv1_w.transpose(1, 2, 3, 0)
    blocks = []
    for i in range(3):
        for j in range(3):
            canvas = jnp.zeros((L, W, H, CNN_CH), jnp.float32)
            canvas = canvas.at[:, 3 * i:3 * i + 5, 3 * j:3 * j + 5, :].set(wt)
            canvas = canvas.reshape(L, W * H, CNN_CH)
            canvas = jnp.pad(canvas, ((0, 0), (0, LP - W * H), (0, 0)))
            blocks.append(canvas.reshape(C_IN, CNN_CH))
    cen = jnp.zeros((C_IN, CNN_CH), jnp.float32)
    cen = cen.at[jnp.arange(L) * LP + 5 * W + 5, jnp.arange(L)].set(
        jnp.ones((L,), jnp.float32))
    blocks.append(cen)
    invr = jnp.repeat(inv, LP)[:, None]
    blocks = [(b * invr).astype(jnp.bfloat16) for b in blocks]
    return jnp.concatenate(blocks, axis=1)


def _dense_body(box_ref, wbig_ref, w2_ref, fc_ref, se_ref,
                b1_ref, b2_ref, bfc_ref, bse_ref,
                a0s_ref, a0c_ref, a1s_ref, a1c_ref, vs_ref, vc_ref,
                a0b_ref, a1b_ref, vb_ref,
                o0_ref, o1_ref, ov_ref):
    f32 = jnp.float32
    box = box_ref[...].astype(jnp.bfloat16)
    acc = jnp.dot(box, wbig_ref[...], preferred_element_type=f32)
    conv1 = jnp.maximum(acc[:, :C_MID] + b1_ref[...], 0.0)
    center = acc[:, C_MID:C_MID + L]
    selff = jnp.maximum(
        jnp.dot(center, se_ref[...], preferred_element_type=f32)
        + bse_ref[...], 0.0)
    h2 = jnp.maximum(
        jnp.dot(conv1, w2_ref[...], preferred_element_type=f32)
        + b2_ref[...], 0.0)
    cnn = jnp.maximum(
        jnp.dot(h2, fc_ref[...], preferred_element_type=f32)
        + bfc_ref[...], 0.0)
    o0_ref[...] = (jnp.dot(selff, a0s_ref[...], preferred_element_type=f32)
                   + jnp.dot(cnn, a0c_ref[...], preferred_element_type=f32)
                   + a0b_ref[...])
    o1_ref[...] = (jnp.dot(selff, a1s_ref[...], preferred_element_type=f32)
                   + jnp.dot(cnn, a1c_ref[...], preferred_element_type=f32)
                   + a1b_ref[...])
    ov_ref[...] = (jnp.dot(selff, vs_ref[...], preferred_element_type=f32)
                   + jnp.dot(cnn, vc_ref[...], preferred_element_type=f32)
                   + vb_ref[...])


def _dense_stage(box, wbig, w2r, fc_t, se_t, b1r, conv2_b, fc_b, se_b,
                 a0s, a0c, a1s, a1c, vs, vc, a0_b, a1_b, v_b, interpret=False):
    nb = B // BB
    full = lambda shp: pl.BlockSpec(shp, lambda i: (0,) * len(shp))
    grid_spec = pl.GridSpec(
        grid=(nb,),
        in_specs=[
            pl.BlockSpec((BB, C_IN), lambda i: (i, 0)),
            full((C_IN, N_COLS)), full((C_MID, CNN_CH)),
            full((CNN_CH, HID // 2)), full((L, HID // 2)),
            full((1, C_MID)), full((1, CNN_CH)), full((1, HID // 2)),
            full((1, HID // 2)),
            full((HID // 2, 128)), full((HID // 2, 128)),
            full((HID // 2, 256)), full((HID // 2, 256)),
            full((HID // 2, 128)), full((HID // 2, 128)),
            full((1, 128)), full((1, 256)), full((1, 128)),
        ],
        out_specs=[
            pl.BlockSpec((BB, 128), lambda i: (i, 0)),
            pl.BlockSpec((BB, 256), lambda i: (i, 0)),
            pl.BlockSpec((BB, 128), lambda i: (i, 0)),
        ],
    )
    o0, o1, ov = pl.pallas_call(
        _dense_body,
        grid_spec=grid_spec,
        out_shape=[
            jax.ShapeDtypeStruct((B, 128), jnp.float32),
            jax.ShapeDtypeStruct((B, 256), jnp.float32),
            jax.ShapeDtypeStruct((B, 128), jnp.float32),
        ],
        compiler_params=pltpu.CompilerParams(
            dimension_semantics=("parallel",)),
        interpret=interpret,
    )(box, wbig, w2r, fc_t, se_t, b1r, conv2_b, fc_b, se_b,
      a0s, a0c, a1s, a1c, vs, vc, a0_b, a1_b, v_b)
    return o0[:, :9], o1, ov[:, :1]


def kernel(observations, conv1_w, conv1_b, conv2_w, conv2_b, fc_w, fc_b,
           se_w, se_b, a0_w, a0_b, a1_w, a1_b, v_w, v_b, max_vec):
    obs_t = jnp.transpose(observations, (2, 1, 0))
    box = _sc_scatter(obs_t)

    wbig = _build_wbig(conv1_w, max_vec)
    w2r = conv2_w.transpose(2, 3, 1, 0).reshape(C_MID, CNN_CH)
    fc_t = fc_w.T
    se_t = se_w.T
    b1r = jnp.tile(conv1_b, NPOS).reshape(1, C_MID)
    pad_h = lambda w, n: jnp.pad(w.T, ((0, 0), (0, n - w.shape[0])))
    a0s, a0c = pad_h(a0_w[:, :256], 128), pad_h(a0_w[:, 256:], 128)
    a1s, a1c = a1_w[:, :256].T, a1_w[:, 256:].T
    vs, vc = pad_h(v_w[:, :256], 128), pad_h(v_w[:, 256:], 128)
    a0b = jnp.pad(a0_b, (0, 128 - 9)).reshape(1, 128)
    a1b = a1_b.reshape(1, 256)
    vb = jnp.pad(v_b, (0, 127)).reshape(1, 128)

    return _dense_stage(box, wbig, w2r, fc_t, se_t, b1r,
                        conv2_b.reshape(1, CNN_CH), fc_b.reshape(1, HID // 2),
                        se_b.reshape(1, HID // 2),
                        a0s, a0c, a1s, a1c, vs, vc, a0b, a1b, vb)


# speedup vs baseline: 1.4981x; 1.0366x over previous
import functools

import jax
import jax.numpy as jnp
from jax import lax
from jax.experimental import pallas as pl
from jax.experimental.pallas import tpu as pltpu
from jax.experimental.pallas import tpu_sc as plsc

B, M, L, W, H = 4096, 200, 24, 11, 11
CNN_CH, HID = 128, 512
LP = 128
C_IN = L * LP
NPOS = 9
C_MID = CNN_CH * NPOS
BB = 512
N_COLS = 10 * CNN_CH

NC, NS, NW = 2, 16, 32
SPW = B // NW
G = 8
NG = SPW // G


def _sc_scatter(obs_t):
    mesh = plsc.VectorSubcoreMesh(core_axis_name="c", subcore_axis_name="s")

    @functools.partial(
        pl.kernel,
        out_type=jax.ShapeDtypeStruct((B, C_IN), jnp.float32),
        mesh=mesh,
        scratch_types=[
            pltpu.VMEM((3, M, SPW), jnp.int32),
            pltpu.VMEM((2, G, C_IN), jnp.float32),
            pltpu.SemaphoreType.DMA,
            pltpu.SemaphoreType.DMA,
        ],
        compiler_params=pltpu.CompilerParams(needs_layout_passes=False),
    )
    def sck(obs_hbm, box_hbm, obs_v, box_v, osem0, osem1):
        wid = lax.axis_index("s") * NC + lax.axis_index("c")
        base = wid * SPW
        lane = lax.iota(jnp.int32, 16)
        pat_s = lane & 7
        pat_t = lane >> 3
        fvec = lane * 0
        zv = jnp.zeros((16,), jnp.float32)

        pltpu.sync_copy(obs_hbm.at[:, :, pl.ds(base, SPW)], obs_v)

        def group(g, p, osem, not_first):
            @pl.when(not_first)
            def _():
                pltpu.make_async_copy(
                    box_v.at[p], box_hbm.at[pl.ds(base, G)], osem).wait()

            def zero(i, carry):
                for r in range(G):
                    box_v[p, r, pl.ds(i * 16, 16)] = zv
                return carry

            lax.fori_loop(0, C_IN // 16, zero, 0, unroll=8)
            s0 = g * G

            def token(t, carry):
                tv = pat_t + 2 * t
                sv = pat_s + s0
                cb = plsc.load_gather(obs_v, [fvec, tv, sv])
                at = plsc.load_gather(obs_v, [fvec + 1, tv, sv])
                vl = plsc.load_gather(obs_v, [fvec + 2, tv, sv])
                cb = jnp.where(cb == 255, 0, cb)
                at = jnp.where(at == 255, 0, at)
                vl = jnp.where(vl == 255, 0, vl)
                x = (cb >> 4) & 15
                y = cb & 15
                valid = (x < W) & (y < H) & (at < L)
                c = jnp.where(valid, at * LP + x * W + y, 0)
                plsc.store_scatter(box_v.at[p], [pat_s, c],
                                   vl.astype(jnp.float32), mask=valid)
                return carry

            lax.fori_loop(0, M // 2, token, 0, unroll=4)
            pltpu.make_async_copy(
                box_v.at[p], box_hbm.at[pl.ds(base + g * G, G)], osem).start()

        def pairs(u, carry):
            group(2 * u, 0, osem0, u > 0)
            group(2 * u + 1, 1, osem1, u > 0)
            return carry

        lax.fori_loop(0, NG // 2, pairs, 0, unroll=False)
        pltpu.make_async_copy(
            box_v.at[0], box_hbm.at[pl.ds(base, G)], osem0).wait()
        pltpu.make_async_copy(
            box_v.at[1], box_hbm.at[pl.ds(base, G)], osem1).wait()

    return sck(obs_t)


def _build_wbig(conv1_w, max_vec):
    inv = 1.0 / (max_vec.reshape(L) + 1e-8)
    wt = conv1_w.transpose(1, 2, 3, 0)
    blocks = []
    for i in range(3):
        for j in range(3):
            canvas = jnp.zeros((L, W, H, CNN_CH), jnp.float32)
            canvas = canvas.at[:, 3 * i:3 * i + 5, 3 * j:3 * j + 5, :].set(wt)
            canvas = canvas.reshape(L, W * H, CNN_CH)
            canvas = jnp.pad(canvas, ((0, 0), (0, LP - W * H), (0, 0)))
            blocks.append(canvas.reshape(C_IN, CNN_CH))
    cen = jnp.zeros((C_IN, CNN_CH), jnp.float32)
    cen = cen.at[jnp.arange(L) * LP + 5 * W + 5, jnp.arange(L)].set(
        jnp.ones((L,), jnp.float32))
    blocks.append(cen)
    invr = jnp.repeat(inv, LP)[:, None]
    blocks = [(b * invr).astype(jnp.bfloat16) for b in blocks]
    return jnp.concatenate(blocks, axis=1)


def _dense_body(box_ref, wbig_ref, w2_ref, fc_ref, se_ref,
                b1_ref, b2_ref, bfc_ref, bse_ref,
                a0s_ref, a0c_ref, a1s_ref, a1c_ref, vs_ref, vc_ref,
                a0b_ref, a1b_ref, vb_ref,
                o0_ref, o1_ref, ov_ref):
    f32 = jnp.float32
    box = box_ref[...].astype(jnp.bfloat16)
    acc = jnp.dot(box, wbig_ref[...], preferred_element_type=f32)
    conv1 = jnp.maximum(acc[:, :C_MID] + b1_ref[...], 0.0)
    center = acc[:, C_MID:C_MID + L]
    selff = jnp.maximum(
        jnp.dot(center, se_ref[...], preferred_element_type=f32)
        + bse_ref[...], 0.0)
    h2 = jnp.maximum(
        jnp.dot(conv1, w2_ref[...], preferred_element_type=f32)
        + b2_ref[...], 0.0)
    cnn = jnp.maximum(
        jnp.dot(h2, fc_ref[...], preferred_element_type=f32)
        + bfc_ref[...], 0.0)
    o0_ref[...] = (jnp.dot(selff, a0s_ref[...], preferred_element_type=f32)
                   + jnp.dot(cnn, a0c_ref[...], preferred_element_type=f32)
                   + a0b_ref[...])
    o1_ref[...] = (jnp.dot(selff, a1s_ref[...], preferred_element_type=f32)
                   + jnp.dot(cnn, a1c_ref[...], preferred_element_type=f32)
                   + a1b_ref[...])
    ov_ref[...] = (jnp.dot(selff, vs_ref[...], preferred_element_type=f32)
                   + jnp.dot(cnn, vc_ref[...], preferred_element_type=f32)
                   + vb_ref[...])


def _dense_stage(box, wbig, w2r, fc_t, se_t, b1r, conv2_b, fc_b, se_b,
                 a0s, a0c, a1s, a1c, vs, vc, a0_b, a1_b, v_b, interpret=False):
    nb = B // BB
    full = lambda shp: pl.BlockSpec(shp, lambda i: (0,) * len(shp))
    grid_spec = pl.GridSpec(
        grid=(nb,),
        in_specs=[
            pl.BlockSpec((BB, C_IN), lambda i: (i, 0)),
            full((C_IN, N_COLS)), full((C_MID, CNN_CH)),
            full((CNN_CH, HID // 2)), full((L, HID // 2)),
            full((1, C_MID)), full((1, CNN_CH)), full((1, HID // 2)),
            full((1, HID // 2)),
            full((HID // 2, 128)), full((HID // 2, 128)),
            full((HID // 2, 256)), full((HID // 2, 256)),
            full((HID // 2, 128)), full((HID // 2, 128)),
            full((1, 128)), full((1, 256)), full((1, 128)),
        ],
        out_specs=[
            pl.BlockSpec((BB, 128), lambda i: (i, 0)),
            pl.BlockSpec((BB, 256), lambda i: (i, 0)),
            pl.BlockSpec((BB, 128), lambda i: (i, 0)),
        ],
    )
    o0, o1, ov = pl.pallas_call(
        _dense_body,
        grid_spec=grid_spec,
        out_shape=[
            jax.ShapeDtypeStruct((B, 128), jnp.float32),
            jax.ShapeDtypeStruct((B, 256), jnp.float32),
            jax.ShapeDtypeStruct((B, 128), jnp.float32),
        ],
        compiler_params=pltpu.CompilerParams(
            dimension_semantics=("parallel",)),
        interpret=interpret,
    )(box, wbig, w2r, fc_t, se_t, b1r, conv2_b, fc_b, se_b,
      a0s, a0c, a1s, a1c, vs, vc, a0_b, a1_b, v_b)
    return o0[:, :9], o1, ov[:, :1]


def kernel(observations, conv1_w, conv1_b, conv2_w, conv2_b, fc_w, fc_b,
           se_w, se_b, a0_w, a0_b, a1_w, a1_b, v_w, v_b, max_vec):
    obs_t = jnp.transpose(observations, (2, 1, 0))
    box = _sc_scatter(obs_t)

    wbig = _build_wbig(conv1_w, max_vec)
    w2r = conv2_w.transpose(2, 3, 1, 0).reshape(C_MID, CNN_CH)
    fc_t = fc_w.T
    se_t = se_w.T
    b1r = jnp.tile(conv1_b, NPOS).reshape(1, C_MID)
    pad_h = lambda w, n: jnp.pad(w.T, ((0, 0), (0, n - w.shape[0])))
    a0s, a0c = pad_h(a0_w[:, :256], 128), pad_h(a0_w[:, 256:], 128)
    a1s, a1c = a1_w[:, :256].T, a1_w[:, 256:].T
    vs, vc = pad_h(v_w[:, :256], 128), pad_h(v_w[:, 256:], 128)
    a0b = jnp.pad(a0_b, (0, 128 - 9)).reshape(1, 128)
    a1b = a1_b.reshape(1, 256)
    vb = jnp.pad(v_b, (0, 127)).reshape(1, 128)

    return _dense_stage(box, wbig, w2r, fc_t, se_t, b1r,
                        conv2_b.reshape(1, CNN_CH), fc_b.reshape(1, HID // 2),
                        se_b.reshape(1, HID // 2),
                        a0s, a0c, a1s, a1c, vs, vc, a0b, a1b, vb)
